# baseline (speedup 1.0000x reference)
import functools

import jax
import jax.numpy as jnp
from jax import lax
from jax.experimental import pallas as pl
from jax.experimental.pallas import tpu as pltpu
from jax.experimental.pallas import tpu_sc as plsc

D_MODEL = 64
NEG_K = 5
K_TOT = NEG_K + 1

NC = 2
NS = 16
LANES = 16
NW = NC * NS

TBLK = 1024


def _sigmoid(x):
    return 1.0 / (1.0 + jnp.exp(-x))


def _transpose_body(t_ref, out_ref):
    x = t_ref[...]
    r = lax.broadcasted_iota(jnp.int32, (D_MODEL, D_MODEL), 0)
    c = lax.broadcasted_iota(jnp.int32, (D_MODEL, D_MODEL), 1)
    eye = jnp.where(r == c, 1.0, 0.0).astype(jnp.float32)
    y = lax.dot_general(x, eye, (((0,), (0,)), ((), ())),
                        preferred_element_type=jnp.float32)
    out_ref[...] = jnp.concatenate(
        [y[: TBLK // 2, :], y[TBLK // 2:, :]], axis=1)


@jax.jit
def _tc_rowmajor_table(table_t):
    vocab = table_t.shape[1]
    grid = pl.cdiv(vocab, TBLK)
    return pl.pallas_call(
        _transpose_body,
        grid=(grid,),
        in_specs=[pl.BlockSpec((D_MODEL, TBLK), lambda i: (0, i))],
        out_specs=pl.BlockSpec((TBLK // 2, 2 * D_MODEL), lambda i: (i, 0)),
        out_shape=jax.ShapeDtypeStruct((grid * (TBLK // 2), 2 * D_MODEL),
                                       jnp.float32),
    )(table_t)


@functools.partial(jax.jit, static_argnames=("batch",))
def _sc_scores(h_t, tgt, neg, table2, batch):
    bw = batch // NW
    ngrp = bw // LANES
    d2 = 2 * D_MODEL

    mesh = plsc.VectorSubcoreMesh(core_axis_name="c", subcore_axis_name="s")

    @functools.partial(
        pl.kernel,
        mesh=mesh,
        compiler_params=pltpu.CompilerParams(needs_layout_passes=False),
        out_type=[
            jax.ShapeDtypeStruct((batch,), jnp.float32),
            jax.ShapeDtypeStruct((NEG_K * batch,), jnp.float32),
        ],
        scratch_types=[
            pltpu.VMEM((bw, NEG_K), jnp.int32),
            pltpu.VMEM((K_TOT, bw), jnp.int32),
            pltpu.VMEM((K_TOT, bw), jnp.int32),
            pltpu.VMEM((K_TOT * bw, d2), jnp.float32),
            pltpu.VMEM((D_MODEL, bw), jnp.float32),
            pltpu.VMEM((K_TOT, bw), jnp.float32),
            pltpu.SemaphoreType.DMA,
        ],
    )
    def sc_fn(h_t_hbm, tgt_hbm, neg_hbm, table_hbm, pos_hbm, negout_hbm,
              negblk_v, sup_v, half_v, rows_v, h_v, out_v, sem):
        wid = lax.axis_index("s") * NC + lax.axis_index("c")
        base = wid * bw

        pltpu.sync_copy(tgt_hbm.at[pl.ds(base, bw)], sup_v.at[0])
        pltpu.sync_copy(neg_hbm.at[pl.ds(base, bw), :], negblk_v)

        iot = lax.iota(jnp.int32, LANES)
        def _split(v):
            return (((v >> 10) << 9) + (v & 511),
                    (((v >> 9) & 1) << 6))

        for g in range(ngrp):
            sl = pl.ds(g * LANES, LANES)
            lanev = iot + g * LANES
            sup_v[0, sl], half_v[0, sl] = _split(sup_v[0, sl])
            for k in range(NEG_K):
                v = plsc.load_gather(
                    negblk_v, [lanev, jnp.full((LANES,), k, jnp.int32)])
                sup_v[k + 1, sl], half_v[k + 1, sl] = _split(v)

        copies = [
            pltpu.async_copy(table_hbm.at[sup_v.at[k]],
                             rows_v.at[pl.ds(k * bw, bw)], sem)
            for k in range(K_TOT)
        ]
        pltpu.sync_copy(h_t_hbm.at[:, pl.ds(base, bw)], h_v)
        for cp in copies:
            cp.wait()

        for g in range(ngrp):
            l0 = g * LANES
            lanev = iot + l0
            rowis = [iot + (k * bw + l0) for k in range(K_TOT)]
            halfs = [half_v[k, pl.ds(l0, LANES)] for k in range(K_TOT)]

            def dbody(d, accs, lanev=lanev, rowis=rowis, halfs=halfs):
                m = (iot + d) & (D_MODEL - 1)
                hv = plsc.load_gather(h_v, [m, lanev])
                return tuple(
                    accs[k] + hv * plsc.load_gather(
                        rows_v, [rowis[k], halfs[k] + m])
                    for k in range(K_TOT)
                )

            accs = lax.fori_loop(
                0, D_MODEL, dbody,
                tuple(jnp.zeros((LANES,), jnp.float32) for _ in range(K_TOT)))
            for k in range(K_TOT):
                out_v[k, pl.ds(l0, LANES)] = _sigmoid(accs[k])

        pltpu.sync_copy(out_v.at[0], pos_hbm.at[pl.ds(base, bw)])
        for k in range(NEG_K):
            pltpu.sync_copy(out_v.at[k + 1],
                            negout_hbm.at[pl.ds(k * batch + base, bw)])

    return sc_fn(h_t, tgt, neg, table2)


def kernel(h, target_index, neg_index, emb_table):
    batch = h.shape[0]
    table2 = _tc_rowmajor_table(emb_table.T)
    h_t = h.T
    pos, neg_to = _sc_scores(h_t, target_index.astype(jnp.int32),
                             neg_index.astype(jnp.int32), table2, batch)
    pos_out = pos.reshape(batch, 1)
    neg_out = neg_to.reshape(NEG_K, batch).T
    pos_label = jnp.ones((batch, 1), dtype=jnp.float32)
    neg_label = jnp.zeros((batch, NEG_K), dtype=jnp.float32)
    return (pos_out, pos_label, neg_out, neg_label)


# speedup vs baseline: 1.5593x; 1.5593x over previous
import functools

import jax
import jax.numpy as jnp
from jax import lax
from jax.experimental import pallas as pl
from jax.experimental.pallas import tpu as pltpu
from jax.experimental.pallas import tpu_sc as plsc

D_MODEL = 64
NEG_K = 5
K_TOT = NEG_K + 1

NC = 2
NS = 16
LANES = 16
NW = NC * NS

TBLK = 4096


def _sigmoid(x):
    return 1.0 / (1.0 + jnp.exp(-x))


def _transpose_body(t_ref, out_ref):
    x = t_ref[...]
    r = lax.broadcasted_iota(jnp.int32, (D_MODEL, D_MODEL), 0)
    c = lax.broadcasted_iota(jnp.int32, (D_MODEL, D_MODEL), 1)
    eye = jnp.where(r == c, 1.0, 0.0).astype(jnp.float32)
    y = lax.dot_general(x, eye, (((0,), (0,)), ((), ())),
                        preferred_element_type=jnp.float32)
    out_ref[:, :D_MODEL] = y[: TBLK // 2, :]
    out_ref[:, D_MODEL:] = y[TBLK // 2:, :]


@jax.jit
def _tc_rowmajor_table(table_t):
    vocab = table_t.shape[1]
    grid = pl.cdiv(vocab, TBLK)
    return pl.pallas_call(
        _transpose_body,
        grid=(grid,),
        in_specs=[pl.BlockSpec((D_MODEL, TBLK), lambda i: (0, i))],
        out_specs=pl.BlockSpec((TBLK // 2, 2 * D_MODEL), lambda i: (i, 0)),
        out_shape=jax.ShapeDtypeStruct((grid * (TBLK // 2), 2 * D_MODEL),
                                       jnp.float32),
    )(table_t)


@functools.partial(jax.jit, static_argnames=("batch",))
def _sc_scores(h_t, tgt, neg, table2, batch):
    bw = batch // NW
    ngrp = bw // LANES
    d2 = 2 * D_MODEL

    mesh = plsc.VectorSubcoreMesh(core_axis_name="c", subcore_axis_name="s")

    @functools.partial(
        pl.kernel,
        mesh=mesh,
        compiler_params=pltpu.CompilerParams(needs_layout_passes=False),
        out_type=[
            jax.ShapeDtypeStruct((batch,), jnp.float32),
            jax.ShapeDtypeStruct((NEG_K * batch,), jnp.float32),
        ],
        scratch_types=[
            pltpu.VMEM((bw, NEG_K), jnp.int32),
            pltpu.VMEM((K_TOT, bw), jnp.int32),
            pltpu.VMEM((K_TOT, bw), jnp.int32),
            pltpu.VMEM((K_TOT * bw, d2), jnp.float32),
            pltpu.VMEM((D_MODEL, bw), jnp.float32),
            pltpu.VMEM((K_TOT, bw), jnp.float32),
            pltpu.SemaphoreType.DMA,
        ],
    )
    def sc_fn(h_t_hbm, tgt_hbm, neg_hbm, table_hbm, pos_hbm, negout_hbm,
              negblk_v, sup_v, half_v, rows_v, h_v, out_v, sem):
        wid = lax.axis_index("s") * NC + lax.axis_index("c")
        base = wid * bw

        pltpu.sync_copy(tgt_hbm.at[pl.ds(base, bw)], sup_v.at[0])
        pltpu.sync_copy(neg_hbm.at[pl.ds(base, bw), :], negblk_v)

        iot = lax.iota(jnp.int32, LANES)
        bsh = TBLK.bit_length() - 1
        hmask = TBLK // 2 - 1

        def _split(v):
            return (((v >> bsh) << (bsh - 1)) + (v & hmask),
                    (((v >> (bsh - 1)) & 1) << 6))

        for g in range(ngrp):
            sl = pl.ds(g * LANES, LANES)
            lanev = iot + g * LANES
            sup_v[0, sl], half_v[0, sl] = _split(sup_v[0, sl])
            for k in range(NEG_K):
                v = plsc.load_gather(
                    negblk_v, [lanev, jnp.full((LANES,), k, jnp.int32)])
                sup_v[k + 1, sl], half_v[k + 1, sl] = _split(v)

        copies = [
            pltpu.async_copy(table_hbm.at[sup_v.at[k]],
                             rows_v.at[pl.ds(k * bw, bw)], sem)
            for k in range(K_TOT)
        ]
        pltpu.sync_copy(h_t_hbm.at[:, pl.ds(base, bw)], h_v)
        for cp in copies:
            cp.wait()

        for g in range(ngrp):
            l0 = g * LANES
            lanev = iot + l0
            rowis = [iot + (k * bw + l0) for k in range(K_TOT)]
            halfs = [half_v[k, pl.ds(l0, LANES)] for k in range(K_TOT)]

            def dbody(d, accs, lanev=lanev, rowis=rowis, halfs=halfs):
                m = (iot + d) & (D_MODEL - 1)
                hv = plsc.load_gather(h_v, [m, lanev])
                return tuple(
                    accs[k] + hv * plsc.load_gather(
                        rows_v, [rowis[k], halfs[k] + m])
                    for k in range(K_TOT)
                )

            accs = lax.fori_loop(
                0, D_MODEL, dbody,
                tuple(jnp.zeros((LANES,), jnp.float32) for _ in range(K_TOT)))
            for k in range(K_TOT):
                out_v[k, pl.ds(l0, LANES)] = _sigmoid(accs[k])

        pltpu.sync_copy(out_v.at[0], pos_hbm.at[pl.ds(base, bw)])
        for k in range(NEG_K):
            pltpu.sync_copy(out_v.at[k + 1],
                            negout_hbm.at[pl.ds(k * batch + base, bw)])

    return sc_fn(h_t, tgt, neg, table2)


def kernel(h, target_index, neg_index, emb_table):
    batch = h.shape[0]
    table2 = _tc_rowmajor_table(emb_table.T)
    h_t = h.T
    pos, neg_to = _sc_scores(h_t, target_index.astype(jnp.int32),
                             neg_index.astype(jnp.int32), table2, batch)
    pos_out = pos.reshape(batch, 1)
    neg_out = neg_to.reshape(NEG_K, batch).T
    pos_label = jnp.ones((batch, 1), dtype=jnp.float32)
    neg_label = jnp.zeros((batch, NEG_K), dtype=jnp.float32)
    return (pos_out, pos_label, neg_out, neg_label)


# speedup vs baseline: 1.6911x; 1.0846x over previous
import functools

import jax
import jax.numpy as jnp
from jax import lax
from jax.experimental import pallas as pl
from jax.experimental.pallas import tpu as pltpu
from jax.experimental.pallas import tpu_sc as plsc

D_MODEL = 64
NEG_K = 5
K_TOT = NEG_K + 1

NC = 2
NS = 16
LANES = 16
NW = NC * NS

TBLK = 8192


def _sigmoid(x):
    return 1.0 / (1.0 + jnp.exp(-x))


def _transpose_body(t_ref, out_ref):
    x = t_ref[...]
    r = lax.broadcasted_iota(jnp.int32, (D_MODEL, D_MODEL), 0)
    c = lax.broadcasted_iota(jnp.int32, (D_MODEL, D_MODEL), 1)
    eye = jnp.where(r == c, 1.0, 0.0).astype(jnp.float32)
    y = lax.dot_general(x, eye, (((0,), (0,)), ((), ())),
                        preferred_element_type=jnp.float32)
    out_ref[:, :D_MODEL] = y[: TBLK // 2, :]
    out_ref[:, D_MODEL:] = y[TBLK // 2:, :]


@jax.jit
def _tc_rowmajor_table(table_t):
    vocab = table_t.shape[1]
    grid = pl.cdiv(vocab, TBLK)
    return pl.pallas_call(
        _transpose_body,
        grid=(grid,),
        in_specs=[pl.BlockSpec((D_MODEL, TBLK), lambda i: (0, i))],
        out_specs=pl.BlockSpec((TBLK // 2, 2 * D_MODEL), lambda i: (i, 0)),
        out_shape=jax.ShapeDtypeStruct((grid * (TBLK // 2), 2 * D_MODEL),
                                       jnp.float32),
    )(table_t)


@functools.partial(jax.jit, static_argnames=("batch",))
def _sc_scores(h_t, tgt, neg, table2, batch):
    bw = batch // NW
    ngrp = bw // LANES
    d2 = 2 * D_MODEL

    mesh = plsc.VectorSubcoreMesh(core_axis_name="c", subcore_axis_name="s")

    @functools.partial(
        pl.kernel,
        mesh=mesh,
        compiler_params=pltpu.CompilerParams(needs_layout_passes=False),
        out_type=[
            jax.ShapeDtypeStruct((batch,), jnp.float32),
            jax.ShapeDtypeStruct((NEG_K * batch,), jnp.float32),
        ],
        scratch_types=[
            pltpu.VMEM((bw, NEG_K), jnp.int32),
            pltpu.VMEM((K_TOT, bw), jnp.int32),
            pltpu.VMEM((K_TOT, bw), jnp.int32),
            pltpu.VMEM((K_TOT * bw, d2), jnp.float32),
            pltpu.VMEM((D_MODEL, bw), jnp.float32),
            pltpu.VMEM((K_TOT, bw), jnp.float32),
            pltpu.SemaphoreType.DMA,
        ],
    )
    def sc_fn(h_t_hbm, tgt_hbm, neg_hbm, table_hbm, pos_hbm, negout_hbm,
              negblk_v, sup_v, half_v, rows_v, h_v, out_v, sem):
        wid = lax.axis_index("s") * NC + lax.axis_index("c")
        base = wid * bw

        pltpu.sync_copy(tgt_hbm.at[pl.ds(base, bw)], sup_v.at[0])
        pltpu.sync_copy(neg_hbm.at[pl.ds(base, bw), :], negblk_v)

        iot = lax.iota(jnp.int32, LANES)
        bsh = TBLK.bit_length() - 1
        hmask = TBLK // 2 - 1

        def _split(v):
            return (((v >> bsh) << (bsh - 1)) + (v & hmask),
                    (((v >> (bsh - 1)) & 1) << 6))

        for g in range(ngrp):
            sl = pl.ds(g * LANES, LANES)
            lanev = iot + g * LANES
            sup_v[0, sl], half_v[0, sl] = _split(sup_v[0, sl])
            for k in range(NEG_K):
                v = plsc.load_gather(
                    negblk_v, [lanev, jnp.full((LANES,), k, jnp.int32)])
                sup_v[k + 1, sl], half_v[k + 1, sl] = _split(v)

        copies = [
            pltpu.async_copy(table_hbm.at[sup_v.at[k]],
                             rows_v.at[pl.ds(k * bw, bw)], sem)
            for k in range(K_TOT)
        ]
        pltpu.sync_copy(h_t_hbm.at[:, pl.ds(base, bw)], h_v)
        for cp in copies:
            cp.wait()

        for g in range(ngrp):
            l0 = g * LANES
            lanev = iot + l0
            rowis = [iot + (k * bw + l0) for k in range(K_TOT)]
            halfs = [half_v[k, pl.ds(l0, LANES)] for k in range(K_TOT)]

            def dbody(d, accs, lanev=lanev, rowis=rowis, halfs=halfs):
                m = (iot + d) & (D_MODEL - 1)
                hv = plsc.load_gather(h_v, [m, lanev])
                return tuple(
                    accs[k] + hv * plsc.load_gather(
                        rows_v, [rowis[k], halfs[k] + m])
                    for k in range(K_TOT)
                )

            accs = lax.fori_loop(
                0, D_MODEL, dbody,
                tuple(jnp.zeros((LANES,), jnp.float32) for _ in range(K_TOT)))
            for k in range(K_TOT):
                out_v[k, pl.ds(l0, LANES)] = _sigmoid(accs[k])

        pltpu.sync_copy(out_v.at[0], pos_hbm.at[pl.ds(base, bw)])
        for k in range(NEG_K):
            pltpu.sync_copy(out_v.at[k + 1],
                            negout_hbm.at[pl.ds(k * batch + base, bw)])

    return sc_fn(h_t, tgt, neg, table2)


def kernel(h, target_index, neg_index, emb_table):
    batch = h.shape[0]
    table2 = _tc_rowmajor_table(emb_table.T)
    h_t = h.T
    pos, neg_to = _sc_scores(h_t, target_index.astype(jnp.int32),
                             neg_index.astype(jnp.int32), table2, batch)
    pos_out = pos.reshape(batch, 1)
    neg_out = neg_to.reshape(NEG_K, batch).T
    pos_label = jnp.ones((batch, 1), dtype=jnp.float32)
    neg_label = jnp.zeros((batch, NEG_K), dtype=jnp.float32)
    return (pos_out, pos_label, neg_out, neg_label)


# speedup vs baseline: 1.6957x; 1.0027x over previous
import functools

import jax
import jax.numpy as jnp
from jax import lax
from jax.experimental import pallas as pl
from jax.experimental.pallas import tpu as pltpu
from jax.experimental.pallas import tpu_sc as plsc

D_MODEL = 64
NEG_K = 5
K_TOT = NEG_K + 1

NC = 2
NS = 16
LANES = 16
NW = NC * NS

TBLK = 16384


def _sigmoid(x):
    return 1.0 / (1.0 + jnp.exp(-x))


def _transpose_body(t_ref, out_ref):
    x = t_ref[...]
    r = lax.broadcasted_iota(jnp.int32, (D_MODEL, D_MODEL), 0)
    c = lax.broadcasted_iota(jnp.int32, (D_MODEL, D_MODEL), 1)
    eye = jnp.where(r == c, 1.0, 0.0).astype(jnp.float32)
    y = lax.dot_general(x, eye, (((0,), (0,)), ((), ())),
                        preferred_element_type=jnp.float32)
    out_ref[:, :D_MODEL] = y[: TBLK // 2, :]
    out_ref[:, D_MODEL:] = y[TBLK // 2:, :]


@jax.jit
def _tc_rowmajor_table(table_t):
    vocab = table_t.shape[1]
    grid = pl.cdiv(vocab, TBLK)
    return pl.pallas_call(
        _transpose_body,
        grid=(grid,),
        in_specs=[pl.BlockSpec((D_MODEL, TBLK), lambda i: (0, i))],
        out_specs=pl.BlockSpec((TBLK // 2, 2 * D_MODEL), lambda i: (i, 0)),
        out_shape=jax.ShapeDtypeStruct((grid * (TBLK // 2), 2 * D_MODEL),
                                       jnp.float32),
    )(table_t)


@functools.partial(jax.jit, static_argnames=("batch",))
def _sc_scores(h_t, tgt, neg, table2, batch):
    bw = batch // NW
    ngrp = bw // LANES
    d2 = 2 * D_MODEL

    mesh = plsc.VectorSubcoreMesh(core_axis_name="c", subcore_axis_name="s")

    @functools.partial(
        pl.kernel,
        mesh=mesh,
        compiler_params=pltpu.CompilerParams(needs_layout_passes=False),
        out_type=[
            jax.ShapeDtypeStruct((batch,), jnp.float32),
            jax.ShapeDtypeStruct((NEG_K * batch,), jnp.float32),
        ],
        scratch_types=[
            pltpu.VMEM((bw, NEG_K), jnp.int32),
            pltpu.VMEM((K_TOT, bw), jnp.int32),
            pltpu.VMEM((K_TOT, bw), jnp.int32),
            pltpu.VMEM((K_TOT * bw, d2), jnp.float32),
            pltpu.VMEM((D_MODEL, bw), jnp.float32),
            pltpu.VMEM((K_TOT, bw), jnp.float32),
            pltpu.SemaphoreType.DMA,
        ],
    )
    def sc_fn(h_t_hbm, tgt_hbm, neg_hbm, table_hbm, pos_hbm, negout_hbm,
              negblk_v, sup_v, half_v, rows_v, h_v, out_v, sem):
        wid = lax.axis_index("s") * NC + lax.axis_index("c")
        base = wid * bw

        pltpu.sync_copy(tgt_hbm.at[pl.ds(base, bw)], sup_v.at[0])
        pltpu.sync_copy(neg_hbm.at[pl.ds(base, bw), :], negblk_v)

        iot = lax.iota(jnp.int32, LANES)
        bsh = TBLK.bit_length() - 1
        hmask = TBLK // 2 - 1

        def _split(v):
            return (((v >> bsh) << (bsh - 1)) + (v & hmask),
                    (((v >> (bsh - 1)) & 1) << 6))

        for g in range(ngrp):
            sl = pl.ds(g * LANES, LANES)
            lanev = iot + g * LANES
            sup_v[0, sl], half_v[0, sl] = _split(sup_v[0, sl])
            for k in range(NEG_K):
                v = plsc.load_gather(
                    negblk_v, [lanev, jnp.full((LANES,), k, jnp.int32)])
                sup_v[k + 1, sl], half_v[k + 1, sl] = _split(v)

        copies = [
            pltpu.async_copy(table_hbm.at[sup_v.at[k]],
                             rows_v.at[pl.ds(k * bw, bw)], sem)
            for k in range(K_TOT)
        ]
        pltpu.sync_copy(h_t_hbm.at[:, pl.ds(base, bw)], h_v)
        for cp in copies:
            cp.wait()

        for g in range(ngrp):
            l0 = g * LANES
            lanev = iot + l0
            rowis = [iot + (k * bw + l0) for k in range(K_TOT)]
            halfs = [half_v[k, pl.ds(l0, LANES)] for k in range(K_TOT)]

            def dbody(d, accs, lanev=lanev, rowis=rowis, halfs=halfs):
                m = (iot + d) & (D_MODEL - 1)
                hv = plsc.load_gather(h_v, [m, lanev])
                return tuple(
                    accs[k] + hv * plsc.load_gather(
                        rows_v, [rowis[k], halfs[k] + m])
                    for k in range(K_TOT)
                )

            accs = lax.fori_loop(
                0, D_MODEL, dbody,
                tuple(jnp.zeros((LANES,), jnp.float32) for _ in range(K_TOT)))
            for k in range(K_TOT):
                out_v[k, pl.ds(l0, LANES)] = _sigmoid(accs[k])

        pltpu.sync_copy(out_v.at[0], pos_hbm.at[pl.ds(base, bw)])
        for k in range(NEG_K):
            pltpu.sync_copy(out_v.at[k + 1],
                            negout_hbm.at[pl.ds(k * batch + base, bw)])

    return sc_fn(h_t, tgt, neg, table2)


def kernel(h, target_index, neg_index, emb_table):
    batch = h.shape[0]
    table2 = _tc_rowmajor_table(emb_table.T)
    h_t = h.T
    pos, neg_to = _sc_scores(h_t, target_index.astype(jnp.int32),
                             neg_index.astype(jnp.int32), table2, batch)
    pos_out = pos.reshape(batch, 1)
    neg_out = neg_to.reshape(NEG_K, batch).T
    pos_label = jnp.ones((batch, 1), dtype=jnp.float32)
    neg_label = jnp.zeros((batch, NEG_K), dtype=jnp.float32)
    return (pos_out, pos_label, neg_out, neg_label)
